# baseline (speedup 1.0000x reference)
import functools

import jax
import jax.numpy as jnp
from jax import lax
from jax.experimental import pallas as pl
from jax.experimental.pallas import tpu as pltpu
from jax.experimental.pallas import tpu_sc as plsc

B, C = 4, 128
N_FINE = 40962
N_COARSE = 10242
PATCH = 7

L = 16
NROWS = B * C
NF_PAD = 41088
NC_PAD = 10368
NB = 864
N_CHUNKS = NC_PAD // L
ROWS_PER_TILE = NROWS // 32


def _sc_maxpool(xp, v2p, npp):
    mesh = plsc.VectorSubcoreMesh(core_axis_name="c", subcore_axis_name="s")

    @functools.partial(
        pl.kernel,
        mesh=mesh,
        compiler_params=pltpu.CompilerParams(
            needs_layout_passes=False, use_tc_tiling_on_sc=False),
        out_type=jax.ShapeDtypeStruct((B, C, N_COARSE), jnp.float32),
        scratch_types=[
            pltpu.VMEM((PATCH * NC_PAD,), jnp.int32),
        ],
    )
    def k(xp_hbm, v2p_hbm, npp_hbm, out_hbm, idxT):
        wid = lax.axis_index("s") * 2 + lax.axis_index("c")

        def phase1(v2pbuf, npbuf):
            pltpu.sync_copy(v2p_hbm, v2pbuf)
            iot = lax.iota(jnp.int32, L)

            def chunk_body(cc, carry):
                pltpu.sync_copy(npp_hbm.at[pl.ds(cc * NB, NB), :], npbuf)
                for p in range(PATCH):
                    colv = jnp.full((L,), p, jnp.int32)

                    def comp_body(ci, carry2, p=p, colv=colv):
                        rowv = iot + ci * L
                        iv = plsc.load_gather(npbuf, [rowv, colv])
                        fv = plsc.load_gather(v2pbuf, [iv])
                        idxT[pl.ds(p * NC_PAD + cc * NB + ci * L, L)] = fv
                        return carry2

                    lax.fori_loop(0, NB // L, comp_body, 0)
                return carry

            lax.fori_loop(0, NC_PAD // NB, chunk_body, 0)

        pl.run_scoped(phase1,
                      pltpu.VMEM((N_FINE,), jnp.int32),
                      pltpu.VMEM((NB, PATCH), jnp.int32))

        def phase2(xrow, outbuf):
            def row_body(kk, carry):
                r = wid * ROWS_PER_TILE + kk
                b = r // C
                c = lax.rem(r, C)
                pltpu.sync_copy(xp_hbm.at[b, c, :], xrow)

                def chunk_body(ci, carry2):
                    i0 = ci * L
                    g = [plsc.load_gather(
                        xrow, [idxT[pl.ds(p * NC_PAD + i0, L)]])
                        for p in range(PATCH)]
                    m01 = jnp.maximum(g[0], g[1])
                    m23 = jnp.maximum(g[2], g[3])
                    m45 = jnp.maximum(g[4], g[5])
                    acc = jnp.maximum(jnp.maximum(m01, m23),
                                      jnp.maximum(m45, g[6]))
                    outbuf[pl.ds(i0, L)] = acc
                    return carry2

                lax.fori_loop(0, N_CHUNKS, chunk_body, 0)
                pltpu.sync_copy(outbuf.at[pl.ds(0, N_COARSE)],
                                out_hbm.at[b, c, :])
                return carry

            lax.fori_loop(0, ROWS_PER_TILE, row_body, 0)

        pl.run_scoped(phase2,
                      pltpu.VMEM((NF_PAD,), jnp.float32),
                      pltpu.VMEM((NC_PAD,), jnp.float32))

    return k(xp, v2p, npp)


def kernel(x, vertices_to_prev_lvl, neighbor_patches):
    xp = jnp.pad(x, ((0, 0), (0, 0), (0, NF_PAD - N_FINE)))
    npp = jnp.pad(neighbor_patches, ((0, NC_PAD - N_COARSE), (0, 0)))
    return _sc_maxpool(xp, vertices_to_prev_lvl, npp)


# speedup vs baseline: 1.1409x; 1.1409x over previous
import functools

import jax
import jax.numpy as jnp
from jax import lax
from jax.experimental import pallas as pl
from jax.experimental.pallas import tpu as pltpu
from jax.experimental.pallas import tpu_sc as plsc

B, C = 4, 128
N_FINE = 40962
N_COARSE = 10242
PATCH = 7

L = 16
NROWS = B * C
NF_PAD = 41088
NC_PAD = 10368
NB = 864
N_CHUNKS = NC_PAD // L
ROWS_PER_TILE = NROWS // 32


def _sc_maxpool(xp, v2p, npp):
    mesh = plsc.VectorSubcoreMesh(core_axis_name="c", subcore_axis_name="s")

    @functools.partial(
        pl.kernel,
        mesh=mesh,
        compiler_params=pltpu.CompilerParams(
            needs_layout_passes=False, use_tc_tiling_on_sc=False),
        out_type=jax.ShapeDtypeStruct((B, C, NC_PAD), jnp.float32),
        scratch_types=[
            pltpu.VMEM((PATCH * NC_PAD,), jnp.int32),
            pltpu.VMEM_SHARED((PATCH * NC_PAD,), jnp.int32),
        ],
    )
    def k(xp_hbm, v2p_hbm, npp_hbm, out_hbm, idxT, idxT_sh):
        sid = lax.axis_index("s")
        wid = sid * 2 + lax.axis_index("c")

        NUNITS = (NC_PAD // NB) * PATCH

        def phase1(v2pbuf, npbuf):
            pltpu.sync_copy(v2p_hbm, v2pbuf)
            iot = lax.iota(jnp.int32, L)

            def unit_body(t, carry):
                u = sid + 16 * t

                @pl.when(u < NUNITS)
                def _():
                    cc = u // PATCH
                    p = lax.rem(u, PATCH)
                    base = p * NC_PAD + cc * NB
                    pltpu.sync_copy(npp_hbm.at[pl.ds(cc * NB, NB), :], npbuf)
                    colv = jnp.full((L,), 0, jnp.int32) + p

                    def comp_body(ci, carry2):
                        rowv = iot + ci * L
                        iv = plsc.load_gather(npbuf, [rowv, colv])
                        fv = plsc.load_gather(v2pbuf, [iv])
                        idxT[pl.ds(base + ci * L, L)] = fv
                        return carry2

                    lax.fori_loop(0, NB // L, comp_body, 0)
                    pltpu.sync_copy(idxT.at[pl.ds(base, NB)],
                                    idxT_sh.at[pl.ds(base, NB)])
                return carry

            lax.fori_loop(0, (NUNITS + 15) // 16, unit_body, 0)
            plsc.subcore_barrier()
            pltpu.sync_copy(idxT_sh, idxT)

        pl.run_scoped(phase1,
                      pltpu.VMEM((N_FINE,), jnp.int32),
                      pltpu.VMEM((NB, PATCH), jnp.int32))

        def phase2(xrow, outbuf):
            def row_body(kk, carry):
                r = wid * ROWS_PER_TILE + kk
                b = r // C
                c = lax.rem(r, C)
                pltpu.sync_copy(xp_hbm.at[b, c, :], xrow)

                def chunk_body(ci, carry2):
                    i0 = ci * L
                    g = [plsc.load_gather(
                        xrow, [idxT[pl.ds(p * NC_PAD + i0, L)]])
                        for p in range(PATCH)]
                    m01 = jnp.maximum(g[0], g[1])
                    m23 = jnp.maximum(g[2], g[3])
                    m45 = jnp.maximum(g[4], g[5])
                    acc = jnp.maximum(jnp.maximum(m01, m23),
                                      jnp.maximum(m45, g[6]))
                    outbuf[pl.ds(i0, L)] = acc
                    return carry2

                lax.fori_loop(0, N_CHUNKS, chunk_body, 0)
                pltpu.sync_copy(outbuf, out_hbm.at[b, c, :])
                return carry

            lax.fori_loop(0, ROWS_PER_TILE, row_body, 0)

        pl.run_scoped(phase2,
                      pltpu.VMEM((NF_PAD,), jnp.float32),
                      pltpu.VMEM((NC_PAD,), jnp.float32))

    return k(xp, v2p, npp)


def kernel(x, vertices_to_prev_lvl, neighbor_patches):
    xp = jnp.pad(x, ((0, 0), (0, 0), (0, NF_PAD - N_FINE)))
    npp = jnp.pad(neighbor_patches, ((0, NC_PAD - N_COARSE), (0, 0)))
    out_pad = _sc_maxpool(xp, vertices_to_prev_lvl, npp)
    return out_pad[:, :, :N_COARSE]
